# Initial kernel scaffold; baseline (speedup 1.0000x reference)
import jax
import jax.numpy as jnp
from jax.experimental import pallas as pl


def kernel(x, edge_index, batch, shape_emb, color_emb, W_pre, b_pre, W1_rel, b1_rel, W1_root, W2_rel, b2_rel, W2_root, W_cls, b_cls):
    raise NotImplementedError("write your pallas kernel here")


# speedup vs baseline: 8.0471x; 8.0471x over previous
import functools

import jax
import jax.numpy as jnp
from jax import lax
from jax.experimental import pallas as pl
from jax.experimental.pallas import tpu as pltpu
from jax.experimental.pallas import tpu_sc as plsc

N = 100000
E = 1600000
NUM_GRAPHS = 64
NUM_CLASSES = 10

BN = 2000
GRID_N = N // BN
EROWSP = 12504
KB = 8
NBLK = EROWSP // KB
NBF = NBLK // 16
NBR = NBLK - 16 * NBF
NP = 100096
NPT = NP // 16
ZROWS = 368


def _tc_embed_body(x0_ref, x1_ref, se_ref, ce_ref, wp_ref, bp_ref,
                   p0_ref, p1_ref):
    t_s = jnp.dot(se_ref[...], wp_ref[0:8, :],
                  preferred_element_type=jnp.float32)
    t_c = jnp.dot(ce_ref[...], wp_ref[8:16, :],
                  preferred_element_type=jnp.float32)
    x0 = x0_ref[...]
    x1 = x1_ref[...]
    oh_s = (x0 == lax.broadcasted_iota(jnp.int32, (BN, 16), 1)
            ).astype(jnp.float32)
    oh_c = (x1 == lax.broadcasted_iota(jnp.int32, (BN, 8), 1)
            ).astype(jnp.float32)
    h = (jnp.dot(oh_s, t_s, preferred_element_type=jnp.float32)
         + jnp.dot(oh_c, t_c, preferred_element_type=jnp.float32)
         + bp_ref[...])
    h = jnp.maximum(h, 0.0)
    p0_ref[...] = h[:, 0:16]
    p1_ref[...] = h[:, 16:32]


def _tc_embed(x0, x1, se, ce, wp, bp):
    f32 = jnp.float32
    return pl.pallas_call(
        _tc_embed_body,
        grid=(GRID_N,),
        in_specs=[
            pl.BlockSpec((BN, 1), lambda i: (i, 0)),
            pl.BlockSpec((BN, 1), lambda i: (i, 0)),
            pl.BlockSpec((16, 8), lambda i: (0, 0)),
            pl.BlockSpec((8, 8), lambda i: (0, 0)),
            pl.BlockSpec((16, 32), lambda i: (0, 0)),
            pl.BlockSpec((1, 32), lambda i: (0, 0)),
        ],
        out_specs=[pl.BlockSpec((BN, 16), lambda i: (i, 0))] * 2,
        out_shape=[jax.ShapeDtypeStruct((N, 16), f32)] * 2,
    )(x0, x1, se, ce, wp, bp)


def _tc_layer_body(a0_ref, a1_ref, h0_ref, h1_ref, wr_ref, b_ref, wo_ref,
                   o0_ref, o1_ref, o2_ref, o3_ref):
    acc = b_ref[...]
    for p, ref in enumerate((a0_ref, a1_ref)):
        acc = acc + jnp.dot(ref[...], wr_ref[16 * p:16 * (p + 1), :],
                            preferred_element_type=jnp.float32)
    for p, ref in enumerate((h0_ref, h1_ref)):
        acc = acc + jnp.dot(ref[...], wo_ref[16 * p:16 * (p + 1), :],
                            preferred_element_type=jnp.float32)
    h = jnp.maximum(acc, 0.0)
    for p, ref in enumerate((o0_ref, o1_ref, o2_ref, o3_ref)):
        ref[...] = h[:, 16 * p:16 * (p + 1)]


def _tc_layer(aggs, hs, wr, b, wo):
    f32 = jnp.float32
    nspec = pl.BlockSpec((BN, 16), lambda i: (i, 0))
    return pl.pallas_call(
        _tc_layer_body,
        grid=(GRID_N,),
        in_specs=[nspec, nspec, nspec, nspec,
                  pl.BlockSpec((32, 64), lambda i: (0, 0)),
                  pl.BlockSpec((1, 64), lambda i: (0, 0)),
                  pl.BlockSpec((32, 64), lambda i: (0, 0))],
        out_specs=[nspec] * 4,
        out_shape=[jax.ShapeDtypeStruct((N, 16), f32)] * 4,
    )(*aggs, *hs, wr, b, wo)


def _tc_final_body(a0, a1, a2, a3, h0, h1, h2, h3, wr_ref, b_ref, wo_ref,
                   bat_ref, wc_ref, bc_ref, out_ref, psum, pcnt):
    i = pl.program_id(0)

    @pl.when(i == 0)
    def _():
        psum[...] = jnp.zeros_like(psum)
        pcnt[...] = jnp.zeros_like(pcnt)

    acc = b_ref[...]
    for p, ref in enumerate((a0, a1, a2, a3)):
        acc = acc + jnp.dot(ref[...], wr_ref[16 * p:16 * (p + 1), :],
                            preferred_element_type=jnp.float32)
    for p, ref in enumerate((h0, h1, h2, h3)):
        acc = acc + jnp.dot(ref[...], wo_ref[16 * p:16 * (p + 1), :],
                            preferred_element_type=jnp.float32)
    h = jnp.maximum(acc, 0.0)

    oh = (bat_ref[...] == lax.broadcasted_iota(jnp.int32, (BN, NUM_GRAPHS), 1)
          ).astype(jnp.float32)
    psum[...] += lax.dot_general(oh, h, (((0,), (0,)), ((), ())),
                                 preferred_element_type=jnp.float32)
    pcnt[...] += lax.dot_general(
        oh, jnp.ones((BN, 1), jnp.float32), (((0,), (0,)), ((), ())),
        preferred_element_type=jnp.float32)

    @pl.when(i == GRID_N - 1)
    def _():
        pooled = psum[...] / jnp.maximum(pcnt[...], 1.0)
        out_ref[...] = (jnp.dot(pooled, wc_ref[...],
                                preferred_element_type=jnp.float32)
                        + bc_ref[...])


def _tc_final(aggs, hs, wr, b, wo, bat, wc, bc):
    f32 = jnp.float32
    nspec = pl.BlockSpec((BN, 16), lambda i: (i, 0))
    return pl.pallas_call(
        _tc_final_body,
        grid=(GRID_N,),
        in_specs=[nspec] * 8 + [
            pl.BlockSpec((64, 64), lambda i: (0, 0)),
            pl.BlockSpec((1, 64), lambda i: (0, 0)),
            pl.BlockSpec((64, 64), lambda i: (0, 0)),
            pl.BlockSpec((BN, 1), lambda i: (i, 0)),
            pl.BlockSpec((64, NUM_CLASSES), lambda i: (0, 0)),
            pl.BlockSpec((1, NUM_CLASSES), lambda i: (0, 0)),
        ],
        out_specs=pl.BlockSpec((NUM_GRAPHS, NUM_CLASSES), lambda i: (0, 0)),
        out_shape=jax.ShapeDtypeStruct((NUM_GRAPHS, NUM_CLASSES), f32),
        scratch_shapes=[pltpu.VMEM((NUM_GRAPHS, NUM_GRAPHS), f32),
                        pltpu.VMEM((NUM_GRAPHS, 1), f32)],
    )(*aggs, *hs, wr, b, wo, bat, wc, bc)


def _sc_plane(plane_hbm, out_hbm, src_i, dst_i, acc, zbuf, sbuf, dbuf,
              rows, sem, s):
    for z in range(NPT // ZROWS):
        pltpu.sync_copy(zbuf, acc.at[pl.ds(s * NPT + z * ZROWS, ZROWS)])
    plsc.subcore_barrier()

    nblk = NBF + jnp.where(s < NBR, 1, 0)

    def blk(t, carry):
        b = s + t * 16
        pltpu.sync_copy(src_i.at[pl.ds(b * KB, KB)], sbuf)
        pltpu.sync_copy(dst_i.at[pl.ds(b * KB, KB)], dbuf)
        descs = [pltpu.async_copy(plane_hbm.at[sbuf.at[j]], rows[j], sem)
                 for j in range(KB)]
        for d in descs:
            d.wait()
        for j in range(KB):
            pltpu.sync_copy(rows[j], acc.at[dbuf.at[j]], add=True)
        return carry

    lax.fori_loop(0, nblk, blk, 0)
    plsc.subcore_barrier()
    pltpu.sync_copy(acc.at[pl.ds(s * NPT, NPT)],
                    out_hbm.at[pl.ds(s * NPT, NPT)])
    plsc.subcore_barrier()


def _make_sc_segsum(num_planes):
    mesh = plsc.VectorSubcoreMesh(core_axis_name="c", subcore_axis_name="s")
    f32 = jnp.float32

    def body(src_i, dst_i, *refs):
        planes = refs[:num_planes]
        outs = refs[num_planes:2 * num_planes]
        acc, zbuf, sbuf, dbuf = refs[2 * num_planes:2 * num_planes + 4]
        rows = refs[2 * num_planes + 4:2 * num_planes + 4 + KB]
        sem = refs[-1]
        c = lax.axis_index("c")
        s = lax.axis_index("s")

        def zrow(i, carry):
            zbuf[i, :] = jnp.zeros((16,), f32)
            return carry

        lax.fori_loop(0, ZROWS, zrow, 0)

        for p in range(num_planes):
            @pl.when(c == (p % 2))
            def _(p=p):
                _sc_plane(planes[p], outs[p], src_i, dst_i, acc, zbuf,
                          sbuf, dbuf, rows, sem, s)

    fn = pl.kernel(
        body,
        out_type=[jax.ShapeDtypeStruct((NP, 16), f32)] * num_planes,
        mesh=mesh,
        compiler_params=pltpu.CompilerParams(use_tc_tiling_on_sc=False),
        scratch_types=[
            pltpu.VMEM_SHARED((NP, 16), f32),
            pltpu.VMEM((ZROWS, 16), f32),
            pltpu.VMEM((KB, 128), jnp.int32),
            pltpu.VMEM((KB, 128), jnp.int32),
        ] + [pltpu.VMEM((128, 16), f32)] * KB + [pltpu.SemaphoreType.DMA],
    )
    return fn


_sc_segsum2 = _make_sc_segsum(2)
_sc_segsum4 = _make_sc_segsum(4)


def kernel(x, edge_index, batch, shape_emb, color_emb, W_pre, b_pre,
           W1_rel, b1_rel, W1_root, W2_rel, b2_rel, W2_root, W_cls, b_cls):
    x = x.astype(jnp.int32)
    x0 = x[:, 0:1]
    x1 = x[:, 1:2]
    npad = EROWSP * 128 - E
    pad_src = jnp.zeros((npad,), jnp.int32)
    pad_dst = N + (jnp.arange(npad, dtype=jnp.int32) % (NP - N))
    src = jnp.concatenate([edge_index[0].astype(jnp.int32), pad_src]
                          ).reshape(EROWSP, 128)
    dst = jnp.concatenate([edge_index[1].astype(jnp.int32), pad_dst]
                          ).reshape(EROWSP, 128)
    bat = batch.astype(jnp.int32).reshape(N, 1)

    h0 = _tc_embed(x0, x1, shape_emb, color_emb, W_pre,
                   b_pre.reshape(1, 32))
    agg1 = _sc_segsum2(src, dst, *h0)
    h1 = _tc_layer(agg1, h0, W1_rel, b1_rel.reshape(1, 64), W1_root)
    agg2 = _sc_segsum4(src, dst, *h1)
    return _tc_final(agg2, h1, W2_rel, b2_rel.reshape(1, 64), W2_root,
                     bat, W_cls, b_cls.reshape(1, NUM_CLASSES))


# speedup vs baseline: 8.9382x; 1.1107x over previous
import functools

import jax
import jax.numpy as jnp
from jax import lax
from jax.experimental import pallas as pl
from jax.experimental.pallas import tpu as pltpu
from jax.experimental.pallas import tpu_sc as plsc

N = 100000
E = 1600000
NUM_GRAPHS = 64
NUM_CLASSES = 10

BN = 2000
GRID_N = N // BN
EROWSP = 12544
KB = 16
NCHK = EROWSP // KB // 16
NP = 100096
NPT = NP // 16
ZROWS = 368


def _tc_embed_body(x0_ref, x1_ref, se_ref, ce_ref, wp_ref, bp_ref,
                   p0_ref, p1_ref):
    t_s = jnp.dot(se_ref[...], wp_ref[0:8, :],
                  preferred_element_type=jnp.float32)
    t_c = jnp.dot(ce_ref[...], wp_ref[8:16, :],
                  preferred_element_type=jnp.float32)
    x0 = x0_ref[...]
    x1 = x1_ref[...]
    oh_s = (x0 == lax.broadcasted_iota(jnp.int32, (BN, 16), 1)
            ).astype(jnp.float32)
    oh_c = (x1 == lax.broadcasted_iota(jnp.int32, (BN, 8), 1)
            ).astype(jnp.float32)
    h = (jnp.dot(oh_s, t_s, preferred_element_type=jnp.float32)
         + jnp.dot(oh_c, t_c, preferred_element_type=jnp.float32)
         + bp_ref[...])
    h = jnp.maximum(h, 0.0)
    p0_ref[...] = h[:, 0:16]
    p1_ref[...] = h[:, 16:32]


def _tc_embed(x0, x1, se, ce, wp, bp):
    f32 = jnp.float32
    return pl.pallas_call(
        _tc_embed_body,
        grid=(GRID_N,),
        in_specs=[
            pl.BlockSpec((BN, 1), lambda i: (i, 0)),
            pl.BlockSpec((BN, 1), lambda i: (i, 0)),
            pl.BlockSpec((16, 8), lambda i: (0, 0)),
            pl.BlockSpec((8, 8), lambda i: (0, 0)),
            pl.BlockSpec((16, 32), lambda i: (0, 0)),
            pl.BlockSpec((1, 32), lambda i: (0, 0)),
        ],
        out_specs=[pl.BlockSpec((BN, 16), lambda i: (i, 0))] * 2,
        out_shape=[jax.ShapeDtypeStruct((N, 16), f32)] * 2,
    )(x0, x1, se, ce, wp, bp)


def _tc_layer_body(a0_ref, a1_ref, h0_ref, h1_ref, wr_ref, b_ref, wo_ref,
                   o0_ref, o1_ref, o2_ref, o3_ref):
    acc = b_ref[...]
    for p, ref in enumerate((a0_ref, a1_ref)):
        acc = acc + jnp.dot(ref[...], wr_ref[16 * p:16 * (p + 1), :],
                            preferred_element_type=jnp.float32)
    for p, ref in enumerate((h0_ref, h1_ref)):
        acc = acc + jnp.dot(ref[...], wo_ref[16 * p:16 * (p + 1), :],
                            preferred_element_type=jnp.float32)
    h = jnp.maximum(acc, 0.0)
    for p, ref in enumerate((o0_ref, o1_ref, o2_ref, o3_ref)):
        ref[...] = h[:, 16 * p:16 * (p + 1)]


def _tc_layer(aggs, hs, wr, b, wo):
    f32 = jnp.float32
    nspec = pl.BlockSpec((BN, 16), lambda i: (i, 0))
    return pl.pallas_call(
        _tc_layer_body,
        grid=(GRID_N,),
        in_specs=[nspec, nspec, nspec, nspec,
                  pl.BlockSpec((32, 64), lambda i: (0, 0)),
                  pl.BlockSpec((1, 64), lambda i: (0, 0)),
                  pl.BlockSpec((32, 64), lambda i: (0, 0))],
        out_specs=[nspec] * 4,
        out_shape=[jax.ShapeDtypeStruct((N, 16), f32)] * 4,
    )(*aggs, *hs, wr, b, wo)


def _tc_final_body(a0, a1, a2, a3, h0, h1, h2, h3, wr_ref, b_ref, wo_ref,
                   bat_ref, wc_ref, bc_ref, out_ref, psum, pcnt):
    i = pl.program_id(0)

    @pl.when(i == 0)
    def _():
        psum[...] = jnp.zeros_like(psum)
        pcnt[...] = jnp.zeros_like(pcnt)

    acc = b_ref[...]
    for p, ref in enumerate((a0, a1, a2, a3)):
        acc = acc + jnp.dot(ref[...], wr_ref[16 * p:16 * (p + 1), :],
                            preferred_element_type=jnp.float32)
    for p, ref in enumerate((h0, h1, h2, h3)):
        acc = acc + jnp.dot(ref[...], wo_ref[16 * p:16 * (p + 1), :],
                            preferred_element_type=jnp.float32)
    h = jnp.maximum(acc, 0.0)

    oh = (bat_ref[...] == lax.broadcasted_iota(jnp.int32, (BN, NUM_GRAPHS), 1)
          ).astype(jnp.float32)
    psum[...] += lax.dot_general(oh, h, (((0,), (0,)), ((), ())),
                                 preferred_element_type=jnp.float32)
    pcnt[...] += lax.dot_general(
        oh, jnp.ones((BN, 1), jnp.float32), (((0,), (0,)), ((), ())),
        preferred_element_type=jnp.float32)

    @pl.when(i == GRID_N - 1)
    def _():
        pooled = psum[...] / jnp.maximum(pcnt[...], 1.0)
        out_ref[...] = (jnp.dot(pooled, wc_ref[...],
                                preferred_element_type=jnp.float32)
                        + bc_ref[...])


def _tc_final(aggs, hs, wr, b, wo, bat, wc, bc):
    f32 = jnp.float32
    nspec = pl.BlockSpec((BN, 16), lambda i: (i, 0))
    return pl.pallas_call(
        _tc_final_body,
        grid=(GRID_N,),
        in_specs=[nspec] * 8 + [
            pl.BlockSpec((64, 64), lambda i: (0, 0)),
            pl.BlockSpec((1, 64), lambda i: (0, 0)),
            pl.BlockSpec((64, 64), lambda i: (0, 0)),
            pl.BlockSpec((BN, 1), lambda i: (i, 0)),
            pl.BlockSpec((64, NUM_CLASSES), lambda i: (0, 0)),
            pl.BlockSpec((1, NUM_CLASSES), lambda i: (0, 0)),
        ],
        out_specs=pl.BlockSpec((NUM_GRAPHS, NUM_CLASSES), lambda i: (0, 0)),
        out_shape=jax.ShapeDtypeStruct((NUM_GRAPHS, NUM_CLASSES), f32),
        scratch_shapes=[pltpu.VMEM((NUM_GRAPHS, NUM_GRAPHS), f32),
                        pltpu.VMEM((NUM_GRAPHS, 1), f32)],
    )(*aggs, *hs, wr, b, wo, bat, wc, bc)


def _sc_plane(plane_hbm, out_hbm, src_i, dst_i, acc, zbuf, sbuf, dbuf,
              rows, semg, sems0, sems1, s):
    for z in range(NPT // ZROWS):
        pltpu.sync_copy(zbuf, acc.at[pl.ds(s * NPT + z * ZROWS, ZROWS)])
    plsc.subcore_barrier()

    sset = (sems0, sems1)

    def _drain_scat(q):
        for i in range(4):
            pltpu.make_async_copy(plane_hbm.at[pl.ds(0, 128)],
                                  rows[4 * q + i], sset[q]).wait()

    def chunk(k, carry):
        base = (s + k * 16) * KB
        pltpu.sync_copy(src_i.at[pl.ds(base, KB)], sbuf)
        pltpu.sync_copy(dst_i.at[pl.ds(base, KB)], dbuf)
        for h in range(4):
            q = h % 2
            if h < 2:
                @pl.when(k > 0)
                def _(q=q):
                    _drain_scat(q)
            else:
                _drain_scat(q)
            gd = [pltpu.async_copy(plane_hbm.at[sbuf.at[4 * h + i]],
                                   rows[4 * q + i], semg)
                  for i in range(4)]
            for d in gd:
                d.wait()
            for i in range(4):
                pltpu.async_copy(rows[4 * q + i],
                                 acc.at[dbuf.at[4 * h + i]], sset[q],
                                 add=True)
        return carry

    lax.fori_loop(0, NCHK, chunk, 0)
    _drain_scat(0)
    _drain_scat(1)
    plsc.subcore_barrier()
    pltpu.sync_copy(acc.at[pl.ds(s * NPT, NPT)],
                    out_hbm.at[pl.ds(s * NPT, NPT)])
    plsc.subcore_barrier()


def _make_sc_segsum(num_planes):
    mesh = plsc.VectorSubcoreMesh(core_axis_name="c", subcore_axis_name="s")
    f32 = jnp.float32

    def body(src_i, dst_i, *refs):
        planes = refs[:num_planes]
        outs = refs[num_planes:2 * num_planes]
        acc, zbuf, sbuf, dbuf = refs[2 * num_planes:2 * num_planes + 4]
        rows = refs[2 * num_planes + 4:2 * num_planes + 4 + 8]
        semg, sems0, sems1 = refs[-3:]
        c = lax.axis_index("c")
        s = lax.axis_index("s")

        def zrow(i, carry):
            zbuf[i, :] = jnp.zeros((16,), f32)
            return carry

        lax.fori_loop(0, ZROWS, zrow, 0)

        for p in range(num_planes):
            @pl.when(c == (p % 2))
            def _(p=p):
                _sc_plane(planes[p], outs[p], src_i, dst_i, acc, zbuf,
                          sbuf, dbuf, rows, semg, sems0, sems1, s)

    fn = pl.kernel(
        body,
        out_type=[jax.ShapeDtypeStruct((NP, 16), f32)] * num_planes,
        mesh=mesh,
        compiler_params=pltpu.CompilerParams(use_tc_tiling_on_sc=False),
        scratch_types=[
            pltpu.VMEM_SHARED((NP, 16), f32),
            pltpu.VMEM((ZROWS, 16), f32),
            pltpu.VMEM((KB, 128), jnp.int32),
            pltpu.VMEM((KB, 128), jnp.int32),
        ] + [pltpu.VMEM((128, 16), f32)] * 8 + [pltpu.SemaphoreType.DMA] * 3,
    )
    return fn


_sc_segsum2 = _make_sc_segsum(2)
_sc_segsum4 = _make_sc_segsum(4)


def kernel(x, edge_index, batch, shape_emb, color_emb, W_pre, b_pre,
           W1_rel, b1_rel, W1_root, W2_rel, b2_rel, W2_root, W_cls, b_cls):
    x = x.astype(jnp.int32)
    x0 = x[:, 0:1]
    x1 = x[:, 1:2]
    npad = EROWSP * 128 - E
    pad_src = jnp.zeros((npad,), jnp.int32)
    pad_dst = N + (jnp.arange(npad, dtype=jnp.int32) % (NP - N))
    src = jnp.concatenate([edge_index[0].astype(jnp.int32), pad_src]
                          ).reshape(EROWSP, 128)
    dst = jnp.concatenate([edge_index[1].astype(jnp.int32), pad_dst]
                          ).reshape(EROWSP, 128)
    bat = batch.astype(jnp.int32).reshape(N, 1)

    h0 = _tc_embed(x0, x1, shape_emb, color_emb, W_pre,
                   b_pre.reshape(1, 32))
    agg1 = _sc_segsum2(src, dst, *h0)
    h1 = _tc_layer(agg1, h0, W1_rel, b1_rel.reshape(1, 64), W1_root)
    agg2 = _sc_segsum4(src, dst, *h1)
    return _tc_final(agg2, h1, W2_rel, b2_rel.reshape(1, 64), W2_root,
                     bat, W_cls, b_cls.reshape(1, NUM_CLASSES))


# speedup vs baseline: 8.9385x; 1.0000x over previous
import functools

import jax
import jax.numpy as jnp
from jax import lax
from jax.experimental import pallas as pl
from jax.experimental.pallas import tpu as pltpu
from jax.experimental.pallas import tpu_sc as plsc

N = 100000
E = 1600000
NUM_GRAPHS = 64
NUM_CLASSES = 10

BN = 2000
GRID_N = N // BN
EROWSP = 12544
KB = 16
NCHK = EROWSP // KB // 16
NP = 100096
NPT = NP // 16
ZROWS = 368


def _tc_embed_body(x0_ref, x1_ref, se_ref, ce_ref, wp_ref, bp_ref,
                   p0_ref, p1_ref):
    t_s = jnp.dot(se_ref[...], wp_ref[0:8, :],
                  preferred_element_type=jnp.float32)
    t_c = jnp.dot(ce_ref[...], wp_ref[8:16, :],
                  preferred_element_type=jnp.float32)
    x0 = x0_ref[...]
    x1 = x1_ref[...]
    oh_s = (x0 == lax.broadcasted_iota(jnp.int32, (BN, 16), 1)
            ).astype(jnp.float32)
    oh_c = (x1 == lax.broadcasted_iota(jnp.int32, (BN, 8), 1)
            ).astype(jnp.float32)
    h = (jnp.dot(oh_s, t_s, preferred_element_type=jnp.float32)
         + jnp.dot(oh_c, t_c, preferred_element_type=jnp.float32)
         + bp_ref[...])
    h = jnp.maximum(h, 0.0)
    p0_ref[...] = h[:, 0:16]
    p1_ref[...] = h[:, 16:32]


def _tc_embed(x0, x1, se, ce, wp, bp):
    f32 = jnp.float32
    return pl.pallas_call(
        _tc_embed_body,
        grid=(GRID_N,),
        in_specs=[
            pl.BlockSpec((BN, 1), lambda i: (i, 0)),
            pl.BlockSpec((BN, 1), lambda i: (i, 0)),
            pl.BlockSpec((16, 8), lambda i: (0, 0)),
            pl.BlockSpec((8, 8), lambda i: (0, 0)),
            pl.BlockSpec((16, 32), lambda i: (0, 0)),
            pl.BlockSpec((1, 32), lambda i: (0, 0)),
        ],
        out_specs=[pl.BlockSpec((BN, 16), lambda i: (i, 0))] * 2,
        out_shape=[jax.ShapeDtypeStruct((N, 16), f32)] * 2,
    )(x0, x1, se, ce, wp, bp)


def _tc_layer_body(a0_ref, a1_ref, h0_ref, h1_ref, wr_ref, b_ref, wo_ref,
                   o0_ref, o1_ref, o2_ref, o3_ref):
    acc = b_ref[...]
    for p, ref in enumerate((a0_ref, a1_ref)):
        acc = acc + jnp.dot(ref[...], wr_ref[16 * p:16 * (p + 1), :],
                            preferred_element_type=jnp.float32)
    for p, ref in enumerate((h0_ref, h1_ref)):
        acc = acc + jnp.dot(ref[...], wo_ref[16 * p:16 * (p + 1), :],
                            preferred_element_type=jnp.float32)
    h = jnp.maximum(acc, 0.0)
    for p, ref in enumerate((o0_ref, o1_ref, o2_ref, o3_ref)):
        ref[...] = h[:, 16 * p:16 * (p + 1)]


def _tc_layer(aggs, hs, wr, b, wo):
    f32 = jnp.float32
    nspec = pl.BlockSpec((BN, 16), lambda i: (i, 0))
    return pl.pallas_call(
        _tc_layer_body,
        grid=(GRID_N,),
        in_specs=[nspec, nspec, nspec, nspec,
                  pl.BlockSpec((32, 64), lambda i: (0, 0)),
                  pl.BlockSpec((1, 64), lambda i: (0, 0)),
                  pl.BlockSpec((32, 64), lambda i: (0, 0))],
        out_specs=[nspec] * 4,
        out_shape=[jax.ShapeDtypeStruct((N, 16), f32)] * 4,
    )(*aggs, *hs, wr, b, wo)


def _tc_final_body(a0, a1, a2, a3, h0, h1, h2, h3, wr_ref, b_ref, wo_ref,
                   bat_ref, wc_ref, bc_ref, out_ref, psum, pcnt):
    i = pl.program_id(0)

    @pl.when(i == 0)
    def _():
        psum[...] = jnp.zeros_like(psum)
        pcnt[...] = jnp.zeros_like(pcnt)

    acc = b_ref[...]
    for p, ref in enumerate((a0, a1, a2, a3)):
        acc = acc + jnp.dot(ref[...], wr_ref[16 * p:16 * (p + 1), :],
                            preferred_element_type=jnp.float32)
    for p, ref in enumerate((h0, h1, h2, h3)):
        acc = acc + jnp.dot(ref[...], wo_ref[16 * p:16 * (p + 1), :],
                            preferred_element_type=jnp.float32)
    h = jnp.maximum(acc, 0.0)

    oh = (bat_ref[...] == lax.broadcasted_iota(jnp.int32, (BN, NUM_GRAPHS), 1)
          ).astype(jnp.float32)
    psum[...] += lax.dot_general(oh, h, (((0,), (0,)), ((), ())),
                                 preferred_element_type=jnp.float32)
    pcnt[...] += lax.dot_general(
        oh, jnp.ones((BN, 1), jnp.float32), (((0,), (0,)), ((), ())),
        preferred_element_type=jnp.float32)

    @pl.when(i == GRID_N - 1)
    def _():
        pooled = psum[...] / jnp.maximum(pcnt[...], 1.0)
        out_ref[...] = (jnp.dot(pooled, wc_ref[...],
                                preferred_element_type=jnp.float32)
                        + bc_ref[...])


def _tc_final(aggs, hs, wr, b, wo, bat, wc, bc):
    f32 = jnp.float32
    nspec = pl.BlockSpec((BN, 16), lambda i: (i, 0))
    return pl.pallas_call(
        _tc_final_body,
        grid=(GRID_N,),
        in_specs=[nspec] * 8 + [
            pl.BlockSpec((64, 64), lambda i: (0, 0)),
            pl.BlockSpec((1, 64), lambda i: (0, 0)),
            pl.BlockSpec((64, 64), lambda i: (0, 0)),
            pl.BlockSpec((BN, 1), lambda i: (i, 0)),
            pl.BlockSpec((64, NUM_CLASSES), lambda i: (0, 0)),
            pl.BlockSpec((1, NUM_CLASSES), lambda i: (0, 0)),
        ],
        out_specs=pl.BlockSpec((NUM_GRAPHS, NUM_CLASSES), lambda i: (0, 0)),
        out_shape=jax.ShapeDtypeStruct((NUM_GRAPHS, NUM_CLASSES), f32),
        scratch_shapes=[pltpu.VMEM((NUM_GRAPHS, NUM_GRAPHS), f32),
                        pltpu.VMEM((NUM_GRAPHS, 1), f32)],
    )(*aggs, *hs, wr, b, wo, bat, wc, bc)


def _sc_plane(plane_hbm, out_hbm, src_i, dst_i, acc, zbuf, sbuf, dbuf,
              rows, semg, sems0, sems1, s):
    for z in range(NPT // ZROWS):
        pltpu.sync_copy(zbuf, acc.at[pl.ds(s * NPT + z * ZROWS, ZROWS)])
    plsc.subcore_barrier()

    sset = (sems0, sems1)

    def _drain_scat(q):
        for i in range(4):
            pltpu.make_async_copy(plane_hbm.at[pl.ds(0, 128)],
                                  rows[4 * q + i], sset[q]).wait()

    def _fire_gat(h, q):
        for i in range(4):
            pltpu.async_copy(plane_hbm.at[sbuf.at[4 * h + i]],
                             rows[4 * q + i], semg)

    def _drain_gat(q):
        for i in range(4):
            pltpu.make_async_copy(plane_hbm.at[pl.ds(0, 128)],
                                  rows[4 * q + i], semg).wait()

    def _fire_scat(h, q):
        for i in range(4):
            pltpu.async_copy(rows[4 * q + i],
                             acc.at[dbuf.at[4 * h + i]], sset[q], add=True)

    def chunk(k, carry):
        base = (s + k * 16) * KB
        pltpu.sync_copy(src_i.at[pl.ds(base, KB)], sbuf)
        pltpu.sync_copy(dst_i.at[pl.ds(base, KB)], dbuf)
        @pl.when(k > 0)
        def _():
            _drain_scat(0)
        _fire_gat(0, 0)
        for h in range(4):
            q = h % 2
            _drain_gat(q)
            _fire_scat(h, q)
            if h < 3:
                qn = (h + 1) % 2
                if h == 0:
                    @pl.when(k > 0)
                    def _():
                        _drain_scat(1)
                else:
                    _drain_scat(qn)
                _fire_gat(h + 1, qn)
        return carry

    lax.fori_loop(0, NCHK, chunk, 0)
    _drain_scat(0)
    _drain_scat(1)
    plsc.subcore_barrier()
    pltpu.sync_copy(acc.at[pl.ds(s * NPT, NPT)],
                    out_hbm.at[pl.ds(s * NPT, NPT)])
    plsc.subcore_barrier()


def _make_sc_segsum(num_planes):
    mesh = plsc.VectorSubcoreMesh(core_axis_name="c", subcore_axis_name="s")
    f32 = jnp.float32

    def body(src_i, dst_i, *refs):
        planes = refs[:num_planes]
        outs = refs[num_planes:2 * num_planes]
        acc, zbuf, sbuf, dbuf = refs[2 * num_planes:2 * num_planes + 4]
        rows = refs[2 * num_planes + 4:2 * num_planes + 4 + 8]
        semg, sems0, sems1 = refs[-3:]
        c = lax.axis_index("c")
        s = lax.axis_index("s")

        def zrow(i, carry):
            zbuf[i, :] = jnp.zeros((16,), f32)
            return carry

        lax.fori_loop(0, ZROWS, zrow, 0)

        for p in range(num_planes):
            @pl.when(c == (p % 2))
            def _(p=p):
                _sc_plane(planes[p], outs[p], src_i, dst_i, acc, zbuf,
                          sbuf, dbuf, rows, semg, sems0, sems1, s)

    fn = pl.kernel(
        body,
        out_type=[jax.ShapeDtypeStruct((NP, 16), f32)] * num_planes,
        mesh=mesh,
        compiler_params=pltpu.CompilerParams(use_tc_tiling_on_sc=False),
        scratch_types=[
            pltpu.VMEM_SHARED((NP, 16), f32),
            pltpu.VMEM((ZROWS, 16), f32),
            pltpu.VMEM((KB, 128), jnp.int32),
            pltpu.VMEM((KB, 128), jnp.int32),
        ] + [pltpu.VMEM((128, 16), f32)] * 8 + [pltpu.SemaphoreType.DMA] * 3,
    )
    return fn


_sc_segsum2 = _make_sc_segsum(2)
_sc_segsum4 = _make_sc_segsum(4)


def kernel(x, edge_index, batch, shape_emb, color_emb, W_pre, b_pre,
           W1_rel, b1_rel, W1_root, W2_rel, b2_rel, W2_root, W_cls, b_cls):
    x = x.astype(jnp.int32)
    x0 = x[:, 0:1]
    x1 = x[:, 1:2]
    npad = EROWSP * 128 - E
    pad_src = jnp.zeros((npad,), jnp.int32)
    pad_dst = N + (jnp.arange(npad, dtype=jnp.int32) % (NP - N))
    src = jnp.concatenate([edge_index[0].astype(jnp.int32), pad_src]
                          ).reshape(EROWSP, 128)
    dst = jnp.concatenate([edge_index[1].astype(jnp.int32), pad_dst]
                          ).reshape(EROWSP, 128)
    bat = batch.astype(jnp.int32).reshape(N, 1)

    h0 = _tc_embed(x0, x1, shape_emb, color_emb, W_pre,
                   b_pre.reshape(1, 32))
    agg1 = _sc_segsum2(src, dst, *h0)
    h1 = _tc_layer(agg1, h0, W1_rel, b1_rel.reshape(1, 64), W1_root)
    agg2 = _sc_segsum4(src, dst, *h1)
    return _tc_final(agg2, h1, W2_rel, b2_rel.reshape(1, 64), W2_root,
                     bat, W_cls, b_cls.reshape(1, NUM_CLASSES))
